# baseline (speedup 1.0000x reference)
import jax
import jax.numpy as jnp
from jax.experimental import pallas as pl

_EMB = 128
_SEEDS = 50


def _body(rand_ref, randt_ref, row_ref, col_ref):
    row_ref[...] = jnp.zeros_like(row_ref)
    r = rand_ref[...]
    rt = randt_ref[...]
    bsz = r.shape[0]
    lt = rt < r[:, None, :]
    ranks = jnp.sum(lt.astype(jnp.int32), axis=1)
    m = jax.lax.broadcasted_iota(jnp.int32, (bsz, _SEEDS, _EMB), 1)
    col_ref[...] = (ranks[:, None, :] == m).astype(jnp.float32)


def kernel(problems):
    batch_size, job_cnt, machine_cnt = problems.shape
    seed_cnt = max(machine_cnt, _SEEDS)
    rand = jax.random.uniform(
        jax.random.key(42), (batch_size, seed_cnt), dtype=jnp.float32
    )
    rand_p = jnp.pad(rand, ((0, 0), (0, _EMB - seed_cnt)), constant_values=2.0)
    rand_t = rand[:, :, None]
    blk = 32
    grid = (batch_size // blk,)
    row_emb, col_emb = pl.pallas_call(
        _body,
        grid=grid,
        in_specs=[
            pl.BlockSpec((blk, _EMB), lambda i: (i, 0)),
            pl.BlockSpec((blk, seed_cnt, 1), lambda i: (i, 0, 0)),
        ],
        out_specs=[
            pl.BlockSpec((blk, job_cnt, _EMB), lambda i: (i, 0, 0)),
            pl.BlockSpec((blk, machine_cnt, _EMB), lambda i: (i, 0, 0)),
        ],
        out_shape=[
            jax.ShapeDtypeStruct((batch_size, job_cnt, _EMB), jnp.float32),
            jax.ShapeDtypeStruct((batch_size, machine_cnt, _EMB), jnp.float32),
        ],
    )(rand_p, rand_t)
    return (row_emb, col_emb)


# speedup vs baseline: 2.8054x; 2.8054x over previous
import jax
import jax.numpy as jnp
from jax.experimental import pallas as pl

_EMB = 128
_SEEDS = 50


def _ranks_body(randt_ref, rankst_ref):
    rt = randt_ref[...]
    lt = rt[None, :, :] < rt[:, None, :]
    rankst_ref[...] = jnp.sum(lt.astype(jnp.int32), axis=1)


def _stream_body(ranks_ref, row_ref, col_ref):
    row_ref[...] = jnp.zeros_like(row_ref)
    ranks = ranks_ref[...]
    m = jax.lax.broadcasted_iota(jnp.int32, (ranks.shape[0], _SEEDS, _EMB), 1)
    col_ref[...] = (ranks[:, None, :] == m).astype(jnp.float32)


def kernel(problems):
    batch_size, job_cnt, machine_cnt = problems.shape
    seed_cnt = max(machine_cnt, _SEEDS)
    rand = jax.random.uniform(
        jax.random.key(42), (batch_size, seed_cnt), dtype=jnp.float32
    )
    rand_t = rand.T
    ranks_t = pl.pallas_call(
        _ranks_body,
        out_shape=jax.ShapeDtypeStruct((seed_cnt, batch_size), jnp.int32),
    )(rand_t)
    ranks = jnp.pad(
        ranks_t.T, ((0, 0), (0, _EMB - seed_cnt)), constant_values=127
    )
    blk = 32
    grid = (batch_size // blk,)
    row_emb, col_emb = pl.pallas_call(
        _stream_body,
        grid=grid,
        in_specs=[pl.BlockSpec((blk, _EMB), lambda i: (i, 0))],
        out_specs=[
            pl.BlockSpec((blk, job_cnt, _EMB), lambda i: (i, 0, 0)),
            pl.BlockSpec((blk, machine_cnt, _EMB), lambda i: (i, 0, 0)),
        ],
        out_shape=[
            jax.ShapeDtypeStruct((batch_size, job_cnt, _EMB), jnp.float32),
            jax.ShapeDtypeStruct((batch_size, machine_cnt, _EMB), jnp.float32),
        ],
    )(ranks)
    return (row_emb, col_emb)
